# Initial kernel scaffold; baseline (speedup 1.0000x reference)
import jax
import jax.numpy as jnp
from jax.experimental import pallas as pl


def kernel(node_classes, edge_scores, edge_indices, valid_adjacency):
    raise NotImplementedError("write your pallas kernel here")


# speedup vs baseline: 693.7625x; 693.7625x over previous
import functools

import jax
import jax.numpy as jnp
from jax import lax
from jax.experimental import pallas as pl
from jax.experimental.pallas import tpu as pltpu
from jax.experimental.pallas import tpu_sc as plsc

N_NODES = 100000
N_EDGES = 6400000
NC = 2
NS = 16
NW = NC * NS
EPW = N_EDGES // NW
CHUNK = 4000
NCHUNK = EPW // CHUNK
STEPS = CHUNK // 16

_LN2 = 0.6931471805599453


def _fast_log1p(u):
    y = 1.0 + u
    yi = plsc.bitcast(y, jnp.int32)
    yf = yi.astype(jnp.float32) * jnp.float32(1.1920928955078125e-7)
    mi = (yi & jnp.int32(0x007FFFFF)) | jnp.int32(0x3F000000)
    mf = plsc.bitcast(mi, jnp.float32)
    log2y = (yf - jnp.float32(124.22551499)
             - jnp.float32(1.498030302) * mf
             - jnp.float32(1.72587999) / (jnp.float32(0.3520887068) + mf))
    return jnp.float32(_LN2) * log2y


def _edge_loss(x, valid):
    ax = jnp.abs(x)
    u = jnp.exp(-ax)
    sp = jnp.maximum(x, jnp.float32(0.0)) + _fast_log1p(u)
    b = jnp.minimum(sp, jnp.float32(100.0))
    a = jnp.minimum(sp - x, jnp.float32(100.0))
    return b + valid * (a - b)


def _body(nodes_hbm, scores_hbm, ind_hbm, adj_hbm, out_hbm,
          class_tbl, adj_tbl, src_buf, dst_buf, scr_buf):
    cid = lax.axis_index("c")
    sid = lax.axis_index("s")
    wid = cid * NS + sid

    pltpu.sync_copy(nodes_hbm, class_tbl)
    pltpu.sync_copy(adj_hbm, adj_tbl)

    base = wid * EPW

    def chunk_body(ci, acc):
        off = base + ci * CHUNK
        pltpu.sync_copy(ind_hbm.at[pl.ds(off, CHUNK)], src_buf)
        pltpu.sync_copy(ind_hbm.at[pl.ds(N_EDGES + off, CHUNK)], dst_buf)
        pltpu.sync_copy(scores_hbm.at[pl.ds(off, CHUNK)], scr_buf)

        def step(j, acc):
            sl = pl.ds(j * 16, 16)
            sv = src_buf[sl]
            dv = dst_buf[sl]
            scls = plsc.load_gather(class_tbl, [sv])
            dcls = plsc.load_gather(class_tbl, [dv])
            valid = plsc.load_gather(adj_tbl, [scls, dcls])
            x = scr_buf[sl]
            return acc + _edge_loss(x, valid)

        return lax.fori_loop(0, STEPS, step, acc, unroll=4)

    acc = lax.fori_loop(0, NCHUNK, chunk_body, jnp.zeros((16,), jnp.float32))

    scr_buf[pl.ds(0, 16)] = acc
    pltpu.sync_copy(scr_buf.at[pl.ds(0, 16)], out_hbm.at[pl.ds(wid * 16, 16)])


@functools.partial(jax.jit, static_argnames=())
def kernel(node_classes, edge_scores, edge_indices, valid_adjacency):
    assert edge_indices.shape == (2, N_EDGES)
    assert node_classes.shape == (N_NODES,)
    scores_flat = edge_scores.reshape(-1)
    ind_flat = edge_indices.reshape(-1)
    adj16 = jnp.zeros((16, 16), jnp.float32).at[:11, :11].set(valid_adjacency)

    mesh = plsc.VectorSubcoreMesh(core_axis_name="c", subcore_axis_name="s")
    partials = pl.kernel(
        _body,
        out_type=jax.ShapeDtypeStruct((NW * 16,), jnp.float32),
        mesh=mesh,
        scratch_types=[
            pltpu.VMEM((N_NODES,), jnp.int32),
            pltpu.VMEM((16, 16), jnp.float32),
            pltpu.VMEM((CHUNK,), jnp.int32),
            pltpu.VMEM((CHUNK,), jnp.int32),
            pltpu.VMEM((CHUNK,), jnp.float32),
        ],
        compiler_params=pltpu.CompilerParams(needs_layout_passes=False),
    )(node_classes, scores_flat, ind_flat, adj16)

    return jnp.sum(partials) / jnp.float32(N_EDGES)


# speedup vs baseline: 1264.0408x; 1.8220x over previous
import functools

import jax
import jax.numpy as jnp
from jax import lax
from jax.experimental import pallas as pl
from jax.experimental.pallas import tpu as pltpu
from jax.experimental.pallas import tpu_sc as plsc

N_NODES = 100000
N_EDGES = 6400000
NC = 2
NS = 16
NW = NC * NS
EPW = N_EDGES // NW
CHUNK = 4000
NCHUNK = EPW // CHUNK
STEPS = CHUNK // 16

_C0 = 2.2117031200252768e-05
_C1 = 0.9990104466294587
_C2 = -0.4891568472023044
_C3 = 0.28330432451740856
_C4 = -0.13011941539126315
_C5 = 0.03010262501167511


def _edge_loss(x, valid):
    u = jnp.exp(-jnp.abs(x))
    p = jnp.float32(_C5)
    p = p * u + jnp.float32(_C4)
    p = p * u + jnp.float32(_C3)
    p = p * u + jnp.float32(_C2)
    p = p * u + jnp.float32(_C1)
    l1p = p * u + jnp.float32(_C0)
    sp = jnp.maximum(x, jnp.float32(0.0)) + l1p
    return jnp.minimum(sp - valid * x, jnp.float32(100.0))


def _body(nodes_hbm, scores_hbm, ind_hbm, adj_hbm, out_hbm,
          class_tbl, adj_tbl, src_buf, dst_buf, scr_buf, sem0, sem1):
    cid = lax.axis_index("c")
    sid = lax.axis_index("s")
    wid = cid * NS + sid

    pltpu.sync_copy(nodes_hbm, class_tbl)
    pltpu.sync_copy(adj_hbm, adj_tbl)

    base = wid * EPW
    sems = (sem0, sem1)

    def start(ci, slot):
        off = base + ci * CHUNK
        dslc = pl.ds(slot * CHUNK, CHUNK)
        sem = sems[slot]
        pltpu.async_copy(ind_hbm.at[pl.ds(off, CHUNK)], src_buf.at[dslc], sem)
        pltpu.async_copy(ind_hbm.at[pl.ds(N_EDGES + off, CHUNK)],
                         dst_buf.at[dslc], sem)
        pltpu.async_copy(scores_hbm.at[pl.ds(off, CHUNK)],
                         scr_buf.at[dslc], sem)

    def wait(ci, slot):
        off = base + ci * CHUNK
        dslc = pl.ds(slot * CHUNK, CHUNK)
        sem = sems[slot]
        pltpu.make_async_copy(ind_hbm.at[pl.ds(off, CHUNK)],
                              src_buf.at[dslc], sem).wait()
        pltpu.make_async_copy(ind_hbm.at[pl.ds(N_EDGES + off, CHUNK)],
                              dst_buf.at[dslc], sem).wait()
        pltpu.make_async_copy(scores_hbm.at[pl.ds(off, CHUNK)],
                              scr_buf.at[dslc], sem).wait()

    def compute(slot, acc):
        sbase = slot * CHUNK

        def step(j, acc):
            sl = pl.ds(sbase + j * 16, 16)
            sv = src_buf[sl]
            dv = dst_buf[sl]
            scls = plsc.load_gather(class_tbl, [sv])
            dcls = plsc.load_gather(class_tbl, [dv])
            valid = plsc.load_gather(adj_tbl, [scls, dcls])
            x = scr_buf[sl]
            return acc + _edge_loss(x, valid)

        return lax.fori_loop(0, STEPS, step, acc, unroll=8)

    start(0, 0)

    def pair_body(k, acc):
        ci0 = k * 2
        start(ci0 + 1, 1)
        wait(ci0, 0)
        acc = compute(0, acc)

        @pl.when(ci0 + 2 < NCHUNK)
        def _():
            start(ci0 + 2, 0)

        wait(ci0 + 1, 1)
        return compute(1, acc)

    acc = lax.fori_loop(0, NCHUNK // 2, pair_body,
                        jnp.zeros((16,), jnp.float32))

    scr_buf[pl.ds(0, 16)] = acc
    pltpu.sync_copy(scr_buf.at[pl.ds(0, 16)], out_hbm.at[pl.ds(wid * 16, 16)])


@functools.partial(jax.jit, static_argnames=())
def kernel(node_classes, edge_scores, edge_indices, valid_adjacency):
    assert edge_indices.shape == (2, N_EDGES)
    assert node_classes.shape == (N_NODES,)
    scores_flat = edge_scores.reshape(-1)
    ind_flat = edge_indices.reshape(-1)
    adj16 = jnp.zeros((16, 16), jnp.float32).at[:11, :11].set(valid_adjacency)

    mesh = plsc.VectorSubcoreMesh(core_axis_name="c", subcore_axis_name="s")
    partials = pl.kernel(
        _body,
        out_type=jax.ShapeDtypeStruct((NW * 16,), jnp.float32),
        mesh=mesh,
        scratch_types=[
            pltpu.VMEM((N_NODES,), jnp.int32),
            pltpu.VMEM((16, 16), jnp.float32),
            pltpu.VMEM((2 * CHUNK,), jnp.int32),
            pltpu.VMEM((2 * CHUNK,), jnp.int32),
            pltpu.VMEM((2 * CHUNK,), jnp.float32),
            pltpu.SemaphoreType.DMA,
            pltpu.SemaphoreType.DMA,
        ],
        compiler_params=pltpu.CompilerParams(needs_layout_passes=False),
    )(node_classes, scores_flat, ind_flat, adj16)

    return jnp.sum(partials) / jnp.float32(N_EDGES)


# speedup vs baseline: 1300.1092x; 1.0285x over previous
import functools

import jax
import jax.numpy as jnp
from jax import lax
from jax.experimental import pallas as pl
from jax.experimental.pallas import tpu as pltpu
from jax.experimental.pallas import tpu_sc as plsc

N_NODES = 100000
N_EDGES = 6400000
NC = 2
NS = 16
NW = NC * NS
EPW = N_EDGES // NW
CHUNK = 4000
NCHUNK = EPW // CHUNK
STEPS = CHUNK // 16

_C0 = 2.2117031200252768e-05
_C1 = 0.9990104466294587
_C2 = -0.4891568472023044
_C3 = 0.28330432451740856
_C4 = -0.13011941539126315
_C5 = 0.03010262501167511


def _edge_loss(x, valid):
    nax = plsc.bitcast(plsc.bitcast(x, jnp.int32) | jnp.int32(-2147483648),
                       jnp.float32)
    u = jnp.exp(nax)
    p = jnp.float32(_C5)
    p = p * u + jnp.float32(_C4)
    p = p * u + jnp.float32(_C3)
    p = p * u + jnp.float32(_C2)
    p = p * u + jnp.float32(_C1)
    l1p = p * u + jnp.float32(_C0)
    sp = jnp.maximum(x, jnp.float32(0.0)) + l1p
    return jnp.minimum(sp - valid * x, jnp.float32(100.0))


def _body(nodes_hbm, scores_hbm, ind_hbm, adj_hbm, out_hbm,
          class_tbl, adj_tbl, src_buf, dst_buf, scr_buf, sem0, sem1):
    cid = lax.axis_index("c")
    sid = lax.axis_index("s")
    wid = cid * NS + sid

    pltpu.sync_copy(nodes_hbm, class_tbl)
    pltpu.sync_copy(adj_hbm, adj_tbl)

    base = wid * EPW
    sems = (sem0, sem1)

    def start(ci, slot):
        off = base + ci * CHUNK
        dslc = pl.ds(slot * CHUNK, CHUNK)
        sem = sems[slot]
        pltpu.async_copy(ind_hbm.at[pl.ds(off, CHUNK)], src_buf.at[dslc], sem)
        pltpu.async_copy(ind_hbm.at[pl.ds(N_EDGES + off, CHUNK)],
                         dst_buf.at[dslc], sem)
        pltpu.async_copy(scores_hbm.at[pl.ds(off, CHUNK)],
                         scr_buf.at[dslc], sem)

    def wait(ci, slot):
        off = base + ci * CHUNK
        dslc = pl.ds(slot * CHUNK, CHUNK)
        sem = sems[slot]
        pltpu.make_async_copy(ind_hbm.at[pl.ds(off, CHUNK)],
                              src_buf.at[dslc], sem).wait()
        pltpu.make_async_copy(ind_hbm.at[pl.ds(N_EDGES + off, CHUNK)],
                              dst_buf.at[dslc], sem).wait()
        pltpu.make_async_copy(scores_hbm.at[pl.ds(off, CHUNK)],
                              scr_buf.at[dslc], sem).wait()

    def compute(slot, acc):
        sbase = slot * CHUNK

        def one(off):
            sl = pl.ds(off, 16)
            sv = src_buf[sl]
            dv = dst_buf[sl]
            scls = plsc.load_gather(class_tbl, [sv])
            dcls = plsc.load_gather(class_tbl, [dv])
            valid = plsc.load_gather(adj_tbl, [scls, dcls])
            x = scr_buf[sl]
            return _edge_loss(x, valid)

        def step(j, accs):
            a0, a1 = accs
            off = sbase + j * 32
            return a0 + one(off), a1 + one(off + 16)

        return lax.fori_loop(0, STEPS // 2, step, acc, unroll=4)

    start(0, 0)

    def pair_body(k, acc):
        ci0 = k * 2
        start(ci0 + 1, 1)
        wait(ci0, 0)
        acc = compute(0, acc)

        @pl.when(ci0 + 2 < NCHUNK)
        def _():
            start(ci0 + 2, 0)

        wait(ci0 + 1, 1)
        return compute(1, acc)

    zero = jnp.zeros((16,), jnp.float32)
    a0, a1 = lax.fori_loop(0, NCHUNK // 2, pair_body, (zero, zero))

    scr_buf[pl.ds(0, 16)] = a0 + a1
    pltpu.sync_copy(scr_buf.at[pl.ds(0, 16)], out_hbm.at[pl.ds(wid * 16, 16)])


@functools.partial(jax.jit, static_argnames=())
def kernel(node_classes, edge_scores, edge_indices, valid_adjacency):
    assert edge_indices.shape == (2, N_EDGES)
    assert node_classes.shape == (N_NODES,)
    scores_flat = edge_scores.reshape(-1)
    ind_flat = edge_indices.reshape(-1)
    adj16 = jnp.zeros((16, 16), jnp.float32).at[:11, :11].set(valid_adjacency)

    mesh = plsc.VectorSubcoreMesh(core_axis_name="c", subcore_axis_name="s")
    partials = pl.kernel(
        _body,
        out_type=jax.ShapeDtypeStruct((NW * 16,), jnp.float32),
        mesh=mesh,
        scratch_types=[
            pltpu.VMEM((N_NODES,), jnp.int32),
            pltpu.VMEM((16, 16), jnp.float32),
            pltpu.VMEM((2 * CHUNK,), jnp.int32),
            pltpu.VMEM((2 * CHUNK,), jnp.int32),
            pltpu.VMEM((2 * CHUNK,), jnp.float32),
            pltpu.SemaphoreType.DMA,
            pltpu.SemaphoreType.DMA,
        ],
        compiler_params=pltpu.CompilerParams(needs_layout_passes=False),
    )(node_classes, scores_flat, ind_flat, adj16)

    return jnp.sum(partials) / jnp.float32(N_EDGES)
